# baseline (device time: 13642 ns/iter reference)
import jax
import jax.numpy as jnp
from jax import lax
from jax.experimental import pallas as pl
from jax.experimental.pallas import tpu as pltpu

N_DEV = 4
CAP = 160


def _index_math(ids_col, ids_row, v_per):
    T = ids_row.shape[1]

    def body(idc_ref, idr_ref, bid_ref, slot_ref):
        my = lax.axis_index("i")
        owner_c = idc_ref[:, :] // v_per
        owner_r = idr_ref[:, :] // v_per
        same = owner_c == owner_r
        i0 = lax.broadcasted_iota(jnp.int32, (T, T), 0)
        i1 = lax.broadcasted_iota(jnp.int32, (T, T), 1)
        earlier = (same & (i1 < i0)).astype(jnp.int32)
        rank_c = jnp.sum(earlier, axis=1, keepdims=True)
        later = (same & (i0 < i1)).astype(jnp.int32)
        rank_r = jnp.sum(later, axis=0, keepdims=True)
        slot_ref[:, :] = owner_c * CAP + rank_c

        jcap = lax.broadcasted_iota(jnp.int32, (CAP, T), 0)
        B = (jcap == rank_r) & (owner_r == my)
        local_r = idr_ref[:, :] & (v_per - 1)
        bid_ref[:, :] = jnp.sum(
            jnp.where(B, local_r, 0), axis=1, keepdims=True, dtype=jnp.int32
        )

    return pl.pallas_call(
        body,
        out_shape=(
            jax.ShapeDtypeStruct((CAP, 1), jnp.int32),
            jax.ShapeDtypeStruct((T, 1), jnp.int32),
        ),
        in_specs=[
            pl.BlockSpec(memory_space=pltpu.VMEM),
            pl.BlockSpec(memory_space=pltpu.VMEM),
        ],
        out_specs=(
            pl.BlockSpec(memory_space=pltpu.VMEM),
            pl.BlockSpec(memory_space=pltpu.VMEM),
        ),
    )(ids_col, ids_row)


def _broadcast_unpermute(qblock, scale, slot, T):
    _, D = qblock.shape
    S = N_DEV * CAP

    def body(q_ref, sc_ref, slot_ref, out_ref, qbuf, scbuf,
             send_sems, recv_sems):
        my = lax.axis_index("i")

        barrier = pltpu.get_barrier_semaphore()
        for p in range(N_DEV - 1):
            peer = (my + 1 + p) % N_DEV
            pl.semaphore_signal(
                barrier, inc=1, device_id=(peer,),
                device_id_type=pl.DeviceIdType.MESH,
            )
        pl.semaphore_wait(barrier, N_DEV - 1)

        rdmas = []
        for p in range(N_DEV - 1):
            peer = (my + 1 + p) % N_DEV
            rdma = pltpu.make_async_remote_copy(
                src_ref=q_ref.at[:, :],
                dst_ref=qbuf.at[pl.ds(my * CAP, CAP), :],
                send_sem=send_sems.at[p],
                recv_sem=recv_sems.at[2 - p],
                device_id=(peer,),
                device_id_type=pl.DeviceIdType.MESH,
            )
            rdma.start()
            rdma_sc = pltpu.make_async_remote_copy(
                src_ref=sc_ref.at[:, :],
                dst_ref=scbuf.at[pl.ds(my * CAP, CAP), :],
                send_sem=send_sems.at[3 + p],
                recv_sem=recv_sems.at[3 + (2 - p)],
                device_id=(peer,),
                device_id_type=pl.DeviceIdType.MESH,
            )
            rdma_sc.start()
            rdmas.append((rdma, rdma_sc))

        qbuf[pl.ds(my * CAP, CAP), :] = q_ref[:, :]
        scbuf[pl.ds(my * CAP, CAP), :] = sc_ref[:, :]
        s_iota = lax.broadcasted_iota(jnp.int32, (T, S), 1)
        P = (slot_ref[:, :] == s_iota).astype(jnp.float32)

        for rdma, rdma_sc in rdmas:
            rdma.wait()
            rdma_sc.wait()

        qsc = qbuf[:, :].astype(jnp.float32) * scbuf[:, :]
        out_ref[:, :] = jnp.dot(P, qsc, preferred_element_type=jnp.float32)

    return pl.pallas_call(
        body,
        out_shape=jax.ShapeDtypeStruct((T, D), jnp.float32),
        in_specs=[
            pl.BlockSpec(memory_space=pltpu.VMEM),
            pl.BlockSpec(memory_space=pltpu.VMEM),
            pl.BlockSpec(memory_space=pltpu.VMEM),
        ],
        out_specs=pl.BlockSpec(memory_space=pltpu.VMEM),
        scratch_shapes=[
            pltpu.VMEM((S, D), jnp.int8),
            pltpu.VMEM((S, 1), jnp.float32),
            pltpu.SemaphoreType.DMA((2 * (N_DEV - 1),)),
            pltpu.SemaphoreType.DMA((2 * (N_DEV - 1),)),
        ],
        compiler_params=pltpu.CompilerParams(collective_id=0),
    )(qblock, scale, slot)


def kernel(ids, E):
    T = ids.shape[0]
    V_per, _ = E.shape

    block_ids, slot = _index_math(ids[:, None], ids[None, :], V_per)
    block = jnp.take(E, block_ids[:, 0], axis=0)
    amax = jnp.maximum(jnp.max(jnp.abs(block), axis=1, keepdims=True), 1e-30)
    qblock = jnp.clip(jnp.rint(block * (127.0 / amax)), -127, 127).astype(
        jnp.int8
    )
    scale = amax * (1.0 / 127.0)
    return _broadcast_unpermute(qblock, scale, slot, T)


# device time: 10374 ns/iter; 1.3150x vs baseline; 1.3150x over previous
import jax
import jax.numpy as jnp
from jax import lax
from jax.experimental import pallas as pl
from jax.experimental.pallas import tpu as pltpu

N_DEV = 4
CAP = 136


def _index_math(ids_row, v_per):
    T = ids_row.shape[1]

    def body(idr_ref, bid_ref, slot_ref):
        my = lax.axis_index("i")
        owner_r = idr_ref[:, :] // v_per

        oh = lax.broadcasted_iota(jnp.int32, (N_DEV, T), 0) == owner_r
        i0 = lax.broadcasted_iota(jnp.int32, (T, T), 0)
        i1 = lax.broadcasted_iota(jnp.int32, (T, T), 1)
        lstrict = (i0 < i1).astype(jnp.bfloat16)
        C = jnp.dot(
            oh.astype(jnp.bfloat16), lstrict,
            preferred_element_type=jnp.float32,
        ).astype(jnp.int32)
        rank_r = jnp.sum(
            jnp.where(oh, C, 0), axis=0, keepdims=True, dtype=jnp.int32
        )
        slot_ref[:, :] = owner_r * CAP + rank_r

        jcap = lax.broadcasted_iota(jnp.int32, (CAP, T), 0)
        B = (jcap == rank_r) & (owner_r == my)
        local_r = idr_ref[:, :] & (v_per - 1)
        bid_ref[:, :] = jnp.sum(
            jnp.where(B, local_r, 0), axis=1, keepdims=True, dtype=jnp.int32
        )

    return pl.pallas_call(
        body,
        out_shape=(
            jax.ShapeDtypeStruct((CAP, 1), jnp.int32),
            jax.ShapeDtypeStruct((1, T), jnp.int32),
        ),
        in_specs=[pl.BlockSpec(memory_space=pltpu.VMEM)],
        out_specs=(
            pl.BlockSpec(memory_space=pltpu.VMEM),
            pl.BlockSpec(memory_space=pltpu.VMEM),
        ),
    )(ids_row)


def _broadcast_unpermute(block, slot, T):
    _, D = block.shape
    S = N_DEV * CAP

    def body(block_ref, slot_ref, out_ref, slotbuf, send_sems, recv_sems):
        my = lax.axis_index("i")

        barrier = pltpu.get_barrier_semaphore()
        for p in range(N_DEV - 1):
            peer = (my + 1 + p) % N_DEV
            pl.semaphore_signal(
                barrier, inc=1, device_id=(peer,),
                device_id_type=pl.DeviceIdType.MESH,
            )
        pl.semaphore_wait(barrier, N_DEV - 1)

        rdmas = []
        for p in range(N_DEV - 1):
            peer = (my + 1 + p) % N_DEV
            rdma = pltpu.make_async_remote_copy(
                src_ref=block_ref.at[:, :],
                dst_ref=slotbuf.at[pl.ds(my * CAP, CAP), :],
                send_sem=send_sems.at[p],
                recv_sem=recv_sems.at[2 - p],
                device_id=(peer,),
                device_id_type=pl.DeviceIdType.MESH,
            )
            rdma.start()
            rdmas.append(rdma)

        slotbuf[pl.ds(my * CAP, CAP), :] = block_ref[:, :]
        s_iota = lax.broadcasted_iota(jnp.int32, (S, T), 0)
        PT = (slot_ref[:, :] == s_iota).astype(jnp.bfloat16)

        for rdma in rdmas:
            rdma.wait()

        out_ref[:, :] = lax.dot_general(
            PT, slotbuf[:, :],
            dimension_numbers=(((0,), (0,)), ((), ())),
            preferred_element_type=jnp.float32,
        )

    return pl.pallas_call(
        body,
        out_shape=jax.ShapeDtypeStruct((T, D), jnp.float32),
        in_specs=[
            pl.BlockSpec(memory_space=pltpu.VMEM),
            pl.BlockSpec(memory_space=pltpu.VMEM),
        ],
        out_specs=pl.BlockSpec(memory_space=pltpu.VMEM),
        scratch_shapes=[
            pltpu.VMEM((S, D), jnp.bfloat16),
            pltpu.SemaphoreType.DMA((N_DEV - 1,)),
            pltpu.SemaphoreType.DMA((N_DEV - 1,)),
        ],
        compiler_params=pltpu.CompilerParams(collective_id=0),
    )(block, slot)


def kernel(ids, E):
    T = ids.shape[0]
    V_per, _ = E.shape

    block_ids, slot = _index_math(ids[None, :], V_per)
    block = jnp.take(E, block_ids[:, 0], axis=0).astype(jnp.bfloat16)
    return _broadcast_unpermute(block, slot, T)
